# Initial kernel scaffold; baseline (speedup 1.0000x reference)
import jax
import jax.numpy as jnp
from jax.experimental import pallas as pl


def kernel(x, a0, a1, coa2, b1, b2, lstm_w_ih, lstm_w_hh, lstm_b_ih, lstm_b_hh, hbs0_l1_w, hbs0_l1_a, hbns01_l1_ws, hbns01_l1_wt, hbns01_l1_a, hbns12_l1_ws, hbns12_l1_wt, hbns12_l1_a, hbs0_l2_w, hbs0_l2_a, hbns01_l2_ws, hbns01_l2_wt, hbns01_l2_a, hbs1_l2_w, hbs1_l2_a, hbns12_l2_ws, hbns12_l2_wt, hbns12_l2_a, hbs2_l2_w, hbs2_l2_a, dec_w1, dec_b1, dec_w2, dec_b2):
    raise NotImplementedError("write your pallas kernel here")


# speedup vs baseline: 2.4385x; 2.4385x over previous
import jax
import jax.numpy as jnp
from jax.experimental import pallas as pl

H = 64
NEG = 0.2
N0 = 325
N1 = 1300
WIN = 12
PRED = 12
LSTM_CHUNK = 1040


def _lstm_kernel(xs_ref, w_ihT_ref, w_hhT_ref, b_ref, h_ref):
    xs = xs_ref[...]
    w_ihT = w_ihT_ref[...]
    w_hhT = w_hhT_ref[...]
    b = b_ref[...]
    rows = xs.shape[0]
    h = jnp.zeros((rows, H), jnp.float32)
    c = jnp.zeros((rows, H), jnp.float32)
    for t in range(WIN):
        xt = xs[:, t:t + 1]
        g = xt * w_ihT + h @ w_hhT + b
        i = g[:, :H]
        f = g[:, H:2 * H]
        gg = g[:, 2 * H:3 * H]
        o = g[:, 3 * H:]
        c = jax.nn.sigmoid(f) * c + jax.nn.sigmoid(i) * jnp.tanh(gg)
        h = jax.nn.sigmoid(o) * jnp.tanh(c)
    h_ref[...] = h


def _row_softmax(e, mask):
    e = jnp.where(mask > 0, e, -1e9)
    e = e - jnp.max(e, axis=1, keepdims=True)
    ex = jnp.where(mask > 0, jnp.exp(e), 0.0)
    return ex / (jnp.sum(ex, axis=1, keepdims=True) + 1e-9)


def _graph_kernel(h_ref, a0_ref, b1_ref,
                  w1_ref, a1v_ref, wt1_ref, an1_ref,
                  w2_ref, a2v_ref, ws2_ref, wt2_ref, an2_ref,
                  dw1_ref, db1_ref, dw2_ref, db2_ref,
                  out_ref):
    x0 = h_ref[0]
    a0 = a0_ref[...]
    b1 = b1_ref[...]

    def hbs(x, w_ref, av_ref):
        w = w_ref[...]
        av = av_ref[...]
        msg = x @ w
        t = msg @ av[:H]
        s = msg @ av[H:]
        e = jax.nn.leaky_relu(t + s.T, NEG)
        attn = _row_softmax(e, a0)
        return jax.nn.relu(attn @ msg)

    x00 = hbs(x0, w1_ref, a1v_ref)

    tm1 = x0 @ wt1_ref[...]
    lt = jax.nn.leaky_relu(tm1 @ an1_ref[...][:H], NEG)
    ecol = jnp.where(b1 > 0, jnp.broadcast_to(lt, (N0, N1)), -1e9)
    cmax = jnp.max(ecol, axis=0, keepdims=True)
    ex = jnp.where(b1 > 0, jnp.exp(ecol - cmax), 0.0)
    alpha1 = ex / (jnp.sum(ex, axis=0, keepdims=True) + 1e-9)
    x1 = jax.nn.relu(
        jax.lax.dot_general(alpha1, tm1, (((0,), (0,)), ((), ()))))

    x002 = hbs(x00, w2_ref, a2v_ref)

    sm = x1 @ ws2_ref[...]
    tm2 = x00 @ wt2_ref[...]
    an2 = an2_ref[...]
    t2 = tm2 @ an2[:H]
    s2 = sm @ an2[H:]
    e2 = jax.nn.leaky_relu(t2 + s2.T, NEG)
    alpha2 = _row_softmax(e2, b1)
    m0f1 = jax.nn.relu(alpha2 @ sm)

    out0 = jax.nn.relu(x002 + m0f1)
    hid = jax.nn.relu(out0 @ dw1_ref[...] + db1_ref[...])
    preds = hid @ dw2_ref[...] + db2_ref[...]
    out_ref[0] = preds


def kernel(x, a0, a1, coa2, b1, b2,
           lstm_w_ih, lstm_w_hh, lstm_b_ih, lstm_b_hh,
           hbs0_l1_w, hbs0_l1_a,
           hbns01_l1_ws, hbns01_l1_wt, hbns01_l1_a,
           hbns12_l1_ws, hbns12_l1_wt, hbns12_l1_a,
           hbs0_l2_w, hbs0_l2_a,
           hbns01_l2_ws, hbns01_l2_wt, hbns01_l2_a,
           hbs1_l2_w, hbs1_l2_a,
           hbns12_l2_ws, hbns12_l2_wt, hbns12_l2_a,
           hbs2_l2_w, hbs2_l2_a,
           dec_w1, dec_b1, dec_w2, dec_b2):
    batch, win, sensors = x.shape
    rows = batch * sensors

    xs = jnp.transpose(x, (0, 2, 1)).reshape(rows, win)
    w_ihT = lstm_w_ih.reshape(1, 4 * H)
    w_hhT = lstm_w_hh.T
    bias = (lstm_b_ih + lstm_b_hh).reshape(1, 4 * H)

    n_chunks = rows // LSTM_CHUNK
    h = pl.pallas_call(
        _lstm_kernel,
        grid=(n_chunks,),
        in_specs=[
            pl.BlockSpec((LSTM_CHUNK, win), lambda i: (i, 0)),
            pl.BlockSpec((1, 4 * H), lambda i: (0, 0)),
            pl.BlockSpec((H, 4 * H), lambda i: (0, 0)),
            pl.BlockSpec((1, 4 * H), lambda i: (0, 0)),
        ],
        out_specs=pl.BlockSpec((LSTM_CHUNK, H), lambda i: (i, 0)),
        out_shape=jax.ShapeDtypeStruct((rows, H), jnp.float32),
    )(xs, w_ihT, w_hhT, bias)
    h = h.reshape(batch, sensors, H)

    bcast = lambda shape: pl.BlockSpec(shape, lambda bidx: tuple(0 for _ in shape))
    out = pl.pallas_call(
        _graph_kernel,
        grid=(batch,),
        in_specs=[
            pl.BlockSpec((1, sensors, H), lambda bidx: (bidx, 0, 0)),
            bcast((N0, N0)),
            bcast((N0, N1)),
            bcast(hbs0_l1_w.shape),
            bcast(hbs0_l1_a.shape),
            bcast(hbns01_l1_wt.shape),
            bcast(hbns01_l1_a.shape),
            bcast(hbs0_l2_w.shape),
            bcast(hbs0_l2_a.shape),
            bcast(hbns01_l2_ws.shape),
            bcast(hbns01_l2_wt.shape),
            bcast(hbns01_l2_a.shape),
            bcast(dec_w1.shape),
            bcast((1, H)),
            bcast(dec_w2.shape),
            bcast((1, PRED)),
        ],
        out_specs=pl.BlockSpec((1, sensors, PRED), lambda bidx: (bidx, 0, 0)),
        out_shape=jax.ShapeDtypeStruct((batch, sensors, PRED), jnp.float32),
    )(h, a0, b1,
      hbs0_l1_w, hbs0_l1_a, hbns01_l1_wt, hbns01_l1_a,
      hbs0_l2_w, hbs0_l2_a, hbns01_l2_ws, hbns01_l2_wt, hbns01_l2_a,
      dec_w1, dec_b1.reshape(1, H), dec_w2, dec_b2.reshape(1, PRED))

    return jnp.transpose(out, (0, 2, 1))


# speedup vs baseline: 2.8092x; 1.1520x over previous
import jax
import jax.numpy as jnp
from jax.experimental import pallas as pl
from jax.experimental.pallas import tpu as pltpu

H = 64
NEG = 0.2
N0 = 325
N1 = 1300
WIN = 12
PRED = 12
LSTM_CHUNK = 1040


def _sig(x):
    return 0.5 + 0.5 * jnp.tanh(0.5 * x)


def _lstm_kernel(xs_ref, w_ihT_ref, w_hhT_ref, b_ref, h_ref):
    xs = xs_ref[...]
    w_ihT = w_ihT_ref[...]
    w_hhT = w_hhT_ref[...]
    b = b_ref[...]
    rows = xs.shape[0]
    h = jnp.zeros((rows, H), jnp.float32)
    c = jnp.zeros((rows, H), jnp.float32)
    for t in range(WIN):
        xt = xs[:, t:t + 1]
        g = xt * w_ihT + h @ w_hhT + b
        i = g[:, :H]
        f = g[:, H:2 * H]
        gg = g[:, 2 * H:3 * H]
        o = g[:, 3 * H:]
        c = _sig(f) * c + _sig(i) * jnp.tanh(gg)
        h = _sig(o) * jnp.tanh(c)
    h_ref[...] = h


def _row_t(vec_ref, lo, mat):
    v = vec_ref[...][lo:lo + H]
    return jax.lax.dot_general(v, mat, (((0,), (1,)), ((), ())))


def _graph_kernel(h_ref, an_ref, b1_ref, bn_ref,
                  w1_ref, a1v_ref, wt1_ref, an1_ref,
                  w2_ref, a2v_ref, ws2_ref, wt2_ref, an2_ref,
                  dw1_ref, db1_ref, dw2_ref, db2_ref,
                  out_ref):
    x0 = h_ref[0]
    aneg = an_ref[...]
    b1 = b1_ref[...]
    bneg = bn_ref[...]

    def hbs(x, w_ref, av_ref):
        msg = x @ w_ref[...]
        t = msg @ av_ref[...][:H]
        s_row = _row_t(av_ref, H, msg)
        ex = jnp.exp(jax.nn.leaky_relu(t + s_row, NEG) + aneg)
        attn = ex / (jnp.sum(ex, axis=1, keepdims=True) + 1e-9)
        return jax.nn.relu(attn @ msg)

    x00 = hbs(x0, w1_ref, a1v_ref)

    tm1 = x0 @ wt1_ref[...]
    lt = jax.nn.leaky_relu(tm1 @ an1_ref[...][:H], NEG)
    w = jnp.exp(lt - jnp.max(lt))
    z = jnp.concatenate([w * tm1, w], axis=1)
    r = jax.lax.dot_general(b1, z, (((0,), (0,)), ((), ())))
    x1 = jax.nn.relu(r[:, :H] / (r[:, H:H + 1] + 1e-9))

    x002 = hbs(x00, w2_ref, a2v_ref)

    sm = x1 @ ws2_ref[...]
    tm2 = x00 @ wt2_ref[...]
    t2 = tm2 @ an2_ref[...][:H]
    s2_row = _row_t(an2_ref, H, sm)
    ex2 = jnp.exp(jax.nn.leaky_relu(t2 + s2_row, NEG) + bneg)
    alpha2 = ex2 / (jnp.sum(ex2, axis=1, keepdims=True) + 1e-9)
    m0f1 = jax.nn.relu(alpha2 @ sm)

    out0 = jax.nn.relu(x002 + m0f1)
    hid = jax.nn.relu(out0 @ dw1_ref[...] + db1_ref[...])
    out_ref[0] = hid @ dw2_ref[...] + db2_ref[...]


def kernel(x, a0, a1, coa2, b1, b2,
           lstm_w_ih, lstm_w_hh, lstm_b_ih, lstm_b_hh,
           hbs0_l1_w, hbs0_l1_a,
           hbns01_l1_ws, hbns01_l1_wt, hbns01_l1_a,
           hbns12_l1_ws, hbns12_l1_wt, hbns12_l1_a,
           hbs0_l2_w, hbs0_l2_a,
           hbns01_l2_ws, hbns01_l2_wt, hbns01_l2_a,
           hbs1_l2_w, hbs1_l2_a,
           hbns12_l2_ws, hbns12_l2_wt, hbns12_l2_a,
           hbs2_l2_w, hbs2_l2_a,
           dec_w1, dec_b1, dec_w2, dec_b2):
    batch, win, sensors = x.shape
    rows = batch * sensors

    xs = jnp.transpose(x, (0, 2, 1)).reshape(rows, win)
    w_ihT = lstm_w_ih.reshape(1, 4 * H)
    w_hhT = lstm_w_hh.T
    bias = (lstm_b_ih + lstm_b_hh).reshape(1, 4 * H)

    n_chunks = rows // LSTM_CHUNK
    h = pl.pallas_call(
        _lstm_kernel,
        grid=(n_chunks,),
        in_specs=[
            pl.BlockSpec((LSTM_CHUNK, win), lambda i: (i, 0)),
            pl.BlockSpec((1, 4 * H), lambda i: (0, 0)),
            pl.BlockSpec((H, 4 * H), lambda i: (0, 0)),
            pl.BlockSpec((1, 4 * H), lambda i: (0, 0)),
        ],
        out_specs=pl.BlockSpec((LSTM_CHUNK, H), lambda i: (i, 0)),
        out_shape=jax.ShapeDtypeStruct((rows, H), jnp.float32),
        compiler_params=pltpu.CompilerParams(
            dimension_semantics=("parallel",)),
    )(xs, w_ihT, w_hhT, bias)
    h = h.reshape(batch, sensors, H)

    aneg = (a0 - 1.0) * 1e9
    bneg = (b1 - 1.0) * 1e9

    bcast = lambda shape: pl.BlockSpec(shape, lambda bidx: tuple(0 for _ in shape))
    out = pl.pallas_call(
        _graph_kernel,
        grid=(batch,),
        in_specs=[
            pl.BlockSpec((1, sensors, H), lambda bidx: (bidx, 0, 0)),
            bcast((N0, N0)),
            bcast((N0, N1)),
            bcast((N0, N1)),
            bcast(hbs0_l1_w.shape),
            bcast(hbs0_l1_a.shape),
            bcast(hbns01_l1_wt.shape),
            bcast(hbns01_l1_a.shape),
            bcast(hbs0_l2_w.shape),
            bcast(hbs0_l2_a.shape),
            bcast(hbns01_l2_ws.shape),
            bcast(hbns01_l2_wt.shape),
            bcast(hbns01_l2_a.shape),
            bcast(dec_w1.shape),
            bcast((1, H)),
            bcast(dec_w2.shape),
            bcast((1, PRED)),
        ],
        out_specs=pl.BlockSpec((1, sensors, PRED), lambda bidx: (bidx, 0, 0)),
        out_shape=jax.ShapeDtypeStruct((batch, sensors, PRED), jnp.float32),
        compiler_params=pltpu.CompilerParams(
            dimension_semantics=("parallel",)),
    )(h, aneg, b1, bneg,
      hbs0_l1_w, hbs0_l1_a, hbns01_l1_wt, hbns01_l1_a,
      hbs0_l2_w, hbs0_l2_a, hbns01_l2_ws, hbns01_l2_wt, hbns01_l2_a,
      dec_w1, dec_b1.reshape(1, H), dec_w2, dec_b2.reshape(1, PRED))

    return jnp.transpose(out, (0, 2, 1))


# speedup vs baseline: 3.1888x; 1.1351x over previous
import jax
import jax.numpy as jnp
from jax.experimental import pallas as pl
from jax.experimental.pallas import tpu as pltpu

H = 64
NEG = 0.2
N0 = 325
N1 = 1300
WIN = 12
PRED = 12
LSTM_CHUNK = 1040


def _sig(x):
    return 0.5 + 0.5 * jnp.tanh(0.5 * x)


def _lstm_kernel(xsa_ref, kmat_ref, w_hhT_ref, h_ref):
    xsa = xsa_ref[...]
    kmat = kmat_ref[...]
    w_hhT = w_hhT_ref[...]
    rows = xsa.shape[0]
    p = xsa @ kmat
    h = jnp.zeros((rows, H), jnp.float32)
    c = jnp.zeros((rows, H), jnp.float32)
    for t in range(WIN):
        g = p[:, t * 4 * H:(t + 1) * 4 * H] + h @ w_hhT
        i = g[:, :H]
        f = g[:, H:2 * H]
        gg = g[:, 2 * H:3 * H]
        o = g[:, 3 * H:]
        c = _sig(f) * c + _sig(i) * jnp.tanh(gg)
        h = _sig(o) * jnp.tanh(c)
    h_ref[...] = h


def _row_t(vec_ref, lo, mat):
    v = vec_ref[...][lo:lo + H]
    return jax.lax.dot_general(v, mat, (((0,), (1,)), ((), ())))


def _graph_kernel(h_ref, an_ref, b1_ref, bn_ref,
                  w1_ref, a1v_ref, wt1_ref, an1_ref,
                  w2_ref, a2v_ref, ws2_ref, wt2_ref, an2_ref,
                  dw1_ref, db1_ref, dw2_ref, db2_ref,
                  out_ref):
    x0 = h_ref[0]
    aneg = an_ref[...]
    b1 = b1_ref[...]
    bneg = bn_ref[...]

    def lrelu(e):
        return jnp.maximum(e, NEG * e)

    def hbs(x, w_ref, av_ref):
        msg = x @ w_ref[...]
        t = msg @ av_ref[...][:H]
        s_row = _row_t(av_ref, H, msg)
        ex = jnp.exp(lrelu(t + s_row) + aneg)
        attn = ex * (1.0 / (jnp.sum(ex, axis=1, keepdims=True) + 1e-9))
        return jax.nn.relu(attn @ msg)

    x00 = hbs(x0, w1_ref, a1v_ref)

    tm1 = x0 @ wt1_ref[...]
    lt = lrelu(tm1 @ an1_ref[...][:H])
    w = jnp.exp(lt - jnp.max(lt))
    z = jnp.concatenate([w * tm1, w], axis=1)
    r = jax.lax.dot_general(b1, z, (((0,), (0,)), ((), ())))
    x1 = jax.nn.relu(r[:, :H] * (1.0 / (r[:, H:H + 1] + 1e-9)))

    x002 = hbs(x00, w2_ref, a2v_ref)

    sm = x1 @ ws2_ref[...]
    tm2 = x00 @ wt2_ref[...]
    t2 = tm2 @ an2_ref[...][:H]
    s2_row = _row_t(an2_ref, H, sm)
    ex2 = jnp.exp(lrelu(t2 + s2_row) + bneg)
    alpha2 = ex2 * (1.0 / (jnp.sum(ex2, axis=1, keepdims=True) + 1e-9))
    m0f1 = jax.nn.relu(alpha2 @ sm)

    out0 = jax.nn.relu(x002 + m0f1)
    hid = jax.nn.relu(out0 @ dw1_ref[...] + db1_ref[...])
    out_ref[0] = hid @ dw2_ref[...] + db2_ref[...]


def kernel(x, a0, a1, coa2, b1, b2,
           lstm_w_ih, lstm_w_hh, lstm_b_ih, lstm_b_hh,
           hbs0_l1_w, hbs0_l1_a,
           hbns01_l1_ws, hbns01_l1_wt, hbns01_l1_a,
           hbns12_l1_ws, hbns12_l1_wt, hbns12_l1_a,
           hbs0_l2_w, hbs0_l2_a,
           hbns01_l2_ws, hbns01_l2_wt, hbns01_l2_a,
           hbs1_l2_w, hbs1_l2_a,
           hbns12_l2_ws, hbns12_l2_wt, hbns12_l2_a,
           hbs2_l2_w, hbs2_l2_a,
           dec_w1, dec_b1, dec_w2, dec_b2):
    batch, win, sensors = x.shape
    rows = batch * sensors

    xs = jnp.transpose(x, (0, 2, 1)).reshape(rows, win)
    xsa = jnp.concatenate([xs, jnp.ones((rows, 1), jnp.float32)], axis=1)
    w_ihT = lstm_w_ih.reshape(1, 4 * H)
    w_hhT = lstm_w_hh.T
    bias = (lstm_b_ih + lstm_b_hh).reshape(1, 4 * H)
    kmat = jnp.concatenate(
        [jnp.kron(jnp.eye(win, dtype=jnp.float32), w_ihT),
         jnp.tile(bias, (1, win))], axis=0)

    n_chunks = rows // LSTM_CHUNK
    h = pl.pallas_call(
        _lstm_kernel,
        grid=(n_chunks,),
        in_specs=[
            pl.BlockSpec((LSTM_CHUNK, win + 1), lambda i: (i, 0)),
            pl.BlockSpec((win + 1, win * 4 * H), lambda i: (0, 0)),
            pl.BlockSpec((H, 4 * H), lambda i: (0, 0)),
        ],
        out_specs=pl.BlockSpec((LSTM_CHUNK, H), lambda i: (i, 0)),
        out_shape=jax.ShapeDtypeStruct((rows, H), jnp.float32),
        compiler_params=pltpu.CompilerParams(
            dimension_semantics=("parallel",)),
    )(xsa, kmat, w_hhT)
    h = h.reshape(batch, sensors, H)

    aneg = (a0 - 1.0) * 1e9
    bneg = (b1 - 1.0) * 1e9

    bcast = lambda shape: pl.BlockSpec(shape, lambda bidx: tuple(0 for _ in shape))
    out = pl.pallas_call(
        _graph_kernel,
        grid=(batch,),
        in_specs=[
            pl.BlockSpec((1, sensors, H), lambda bidx: (bidx, 0, 0)),
            bcast((N0, N0)),
            bcast((N0, N1)),
            bcast((N0, N1)),
            bcast(hbs0_l1_w.shape),
            bcast(hbs0_l1_a.shape),
            bcast(hbns01_l1_wt.shape),
            bcast(hbns01_l1_a.shape),
            bcast(hbs0_l2_w.shape),
            bcast(hbs0_l2_a.shape),
            bcast(hbns01_l2_ws.shape),
            bcast(hbns01_l2_wt.shape),
            bcast(hbns01_l2_a.shape),
            bcast(dec_w1.shape),
            bcast((1, H)),
            bcast(dec_w2.shape),
            bcast((1, PRED)),
        ],
        out_specs=pl.BlockSpec((1, sensors, PRED), lambda bidx: (bidx, 0, 0)),
        out_shape=jax.ShapeDtypeStruct((batch, sensors, PRED), jnp.float32),
        compiler_params=pltpu.CompilerParams(
            dimension_semantics=("parallel",)),
    )(h, aneg, b1, bneg,
      hbs0_l1_w, hbs0_l1_a, hbns01_l1_wt, hbns01_l1_a,
      hbs0_l2_w, hbs0_l2_a, hbns01_l2_ws, hbns01_l2_wt, hbns01_l2_a,
      dec_w1, dec_b1.reshape(1, H), dec_w2, dec_b2.reshape(1, PRED))

    return jnp.transpose(out, (0, 2, 1))
